# baseline (speedup 1.0000x reference)
import functools

import jax
import jax.numpy as jnp
from jax import lax
from jax.experimental import pallas as pl
from jax.experimental.pallas import tpu as pltpu
from jax.experimental.pallas import tpu_sc as plsc

E = 160000
K = 2
D = 128
EK = E * K

NC = 2
NS = 16
NW = NC * NS
PER_W = EK // NW

CR = 200
NBUF = 4
NCHUNK = PER_W // CR
ROUNDS = NCHUNK // NBUF
REM = NCHUNK - ROUNDS * NBUF

FULL_GROUPS = CR // 16
TAIL_ROWS = CR - FULL_GROUPS * 16
CRP = CR if TAIL_ROWS == 0 else (FULL_GROUPS + 1) * 16

assert PER_W % CR == 0 and CR % 2 == 0 and TAIL_ROWS % 2 == 0


def _splat_lane(vec, lane):
    return lax.gather(
        vec,
        jnp.full((16, 1), lane, jnp.int32),
        lax.GatherDimensionNumbers(
            offset_dims=(), collapsed_slice_dims=(0,), start_index_map=(0,)),
        slice_sizes=(1,),
        mode=lax.GatherScatterMode.PROMISE_IN_BOUNDS)


def _do_group(buf, oi_v, w_v, gb, n_edges):
    ovec = oi_v[pl.ds(gb, 16)]
    wvec = w_v[pl.ds(gb, 16)]
    avec = ovec.astype(jnp.float32) * wvec
    for el in range(n_edges):
        l0 = 2 * el
        l1 = l0 + 1
        i0 = gb + l0
        i1 = gb + l1
        w0 = _splat_lane(wvec, l0)
        w1 = _splat_lane(wvec, l1)
        a0 = _splat_lane(avec, l0)
        a1 = _splat_lane(avec, l1)
        for j in range(D // 16):
            sl = pl.ds(j * 16, 16)
            f0 = buf[i0, sl]
            f1 = buf[i1, sl]
            d = f1 - f0
            buf[i0, sl] = f0 * w0 + d * a0
            buf[i1, sl] = f0 * w1 + d * a1


def _compute_chunk(buf, oi_v, w_v):
    @pl.loop(0, FULL_GROUPS)
    def _group(g):
        _do_group(buf, oi_v, w_v, g * 16, 8)

    if TAIL_ROWS:
        _do_group(buf, oi_v, w_v, FULL_GROUPS * 16, TAIL_ROWS // 2)


@functools.partial(
    pl.kernel,
    out_type=jax.ShapeDtypeStruct((EK, D), jnp.float32),
    mesh=plsc.VectorSubcoreMesh(core_axis_name="c", subcore_axis_name="s"),
    scratch_types=(
        [pltpu.VMEM((CR, D), jnp.float32) for _ in range(NBUF)]
        + [pltpu.VMEM((CRP,), jnp.int32) for _ in range(NBUF)]
        + [pltpu.VMEM((CRP,), jnp.float32) for _ in range(NBUF)]
        + [pltpu.SemaphoreType.DMA for _ in range(2 * NBUF)]
    ),
)
def _sc_message(feat_hbm, oi_hbm, w_hbm, out_hbm, *rest):
    bufs = rest[:NBUF]
    ois = rest[NBUF:2 * NBUF]
    ws = rest[2 * NBUF:3 * NBUF]
    ls = rest[3 * NBUF:4 * NBUF]
    ss = rest[4 * NBUF:5 * NBUF]

    wid = lax.axis_index("s") * NC + lax.axis_index("c")
    base = wid * PER_W

    def start_load(cb, p):
        pltpu.async_copy(feat_hbm.at[pl.ds(cb, CR)], bufs[p], ls[p])
        pltpu.async_copy(oi_hbm.at[pl.ds(cb, CR)], ois[p].at[pl.ds(0, CR)],
                         ls[p])
        pltpu.async_copy(w_hbm.at[pl.ds(cb, CR)], ws[p].at[pl.ds(0, CR)],
                         ls[p])

    def wait_load(cb, p):
        pltpu.make_async_copy(feat_hbm.at[pl.ds(cb, CR)], bufs[p],
                              ls[p]).wait()
        pltpu.make_async_copy(oi_hbm.at[pl.ds(cb, CR)],
                              ois[p].at[pl.ds(0, CR)], ls[p]).wait()
        pltpu.make_async_copy(w_hbm.at[pl.ds(cb, CR)],
                              ws[p].at[pl.ds(0, CR)], ls[p]).wait()

    for p in range(NBUF):
        start_load(base + p * CR, p)

    @pl.loop(0, ROUNDS)
    def _round(t):
        c0 = t * NBUF
        for p in range(NBUF):
            cb = base + (c0 + p) * CR
            wait_load(cb, p)
            _compute_chunk(bufs[p], ois[p], ws[p])
            pltpu.async_copy(bufs[p], out_hbm.at[pl.ds(cb, CR)], ss[p])
        for p in range(NBUF):
            cb = base + (c0 + p) * CR
            pltpu.make_async_copy(bufs[p], out_hbm.at[pl.ds(cb, CR)],
                                  ss[p]).wait()
            nb = cb + NBUF * CR
            if p < REM:
                start_load(nb, p)
            else:
                @pl.when(t < ROUNDS - 1)
                def _():
                    start_load(nb, p)

    for p in range(REM):
        cb = base + (ROUNDS * NBUF + p) * CR
        wait_load(cb, p)
        _compute_chunk(bufs[p], ois[p], ws[p])
        pltpu.async_copy(bufs[p], out_hbm.at[pl.ds(cb, CR)], ss[p]).wait()


def kernel(feat_src, orderInfo, edge_weight):
    feat = feat_src.reshape(EK, D)
    oi = orderInfo.reshape(EK).astype(jnp.int32)
    w = edge_weight.reshape(EK)
    return _sc_message(feat, oi, w)
